# baseline (speedup 1.0000x reference)
import functools

import jax
import jax.numpy as jnp
from jax import lax
from jax.experimental import pallas as pl
from jax.experimental.pallas import tpu as pltpu
from jax.experimental.pallas import tpu_sc as plsc


def _scores_body(nodes_ref, w_ref, out_ref):
    x = nodes_ref[0]
    w = w_ref[...]
    s_row = lax.dot_general(w, x, (((0,), (1,)), ((), ())),
                            preferred_element_type=jnp.float32)
    out_ref[0] = s_row


def _scores_call(nodes, W):
    B, N, C = nodes.shape
    return pl.pallas_call(
        _scores_body,
        grid=(B,),
        in_specs=[
            pl.BlockSpec((1, N, C), lambda b: (b, 0, 0)),
            pl.BlockSpec((C, 1), lambda b: (0, 0)),
        ],
        out_specs=pl.BlockSpec((1, 1, N), lambda b: (b, 0, 0)),
        out_shape=jax.ShapeDtypeStruct((B, 1, N), jnp.float32),
    )(nodes, W)


def _topk_body(k, chunk, srow_ref, scol_ref, idxn_ref, idxa_ref):
    b = pl.program_id(0)
    _, N = srow_ref.shape[1], srow_ref.shape[2]
    s_row = srow_ref[0]
    s_col = scol_ref[0]
    r_iota = lax.broadcasted_iota(jnp.int32, (1, k), 1)
    j_iota = lax.broadcasted_iota(jnp.int32, (chunk, N), 1)
    idx_acc = jnp.zeros((1, k), dtype=jnp.int32)
    for ci in range(0, N, chunk):
        sc = s_col[ci:ci + chunk, :]
        i_col = ci + lax.broadcasted_iota(jnp.int32, (chunk, 1), 0)
        before = (s_row > sc) | ((s_row == sc) & (j_iota < i_col))
        rank = jnp.sum(jnp.where(before, 1.0, 0.0), axis=1,
                       keepdims=True).astype(jnp.int32)
        onehot = rank == r_iota
        contrib = jnp.where(onehot, i_col, 0)
        idx_acc = idx_acc + jnp.sum(contrib, axis=0, keepdims=True)
    idx_glob = idx_acc + b * N
    idxn_ref[0] = idx_glob
    idxa_ref[0] = idx_glob * 2


def _topk_call(scores_row, scores_col, k, chunk=256):
    B, _, N = scores_row.shape
    body = functools.partial(_topk_body, k, chunk)
    return pl.pallas_call(
        body,
        grid=(B,),
        in_specs=[
            pl.BlockSpec((1, 1, N), lambda b: (b, 0, 0)),
            pl.BlockSpec((1, N, 1), lambda b: (b, 0, 0)),
        ],
        out_specs=[
            pl.BlockSpec((1, 1, k), lambda b: (b, 0, 0)),
            pl.BlockSpec((1, 1, k), lambda b: (b, 0, 0)),
        ],
        out_shape=[
            jax.ShapeDtypeStruct((B, 1, k), jnp.int32),
            jax.ShapeDtypeStruct((B, 1, k), jnp.int32),
        ],
    )(scores_row, scores_col)


def _gather_call(idxn, idxa, nodes_flat, adj_flat, rows_per_w, ch):
    BK = idxn.shape[0]
    C = nodes_flat.shape[1]
    K = adj_flat.shape[1]
    info = plsc.get_sparse_core_info()
    nc = info.num_cores

    mesh = plsc.VectorSubcoreMesh(core_axis_name="c", subcore_axis_name="s")

    @functools.partial(
        pl.kernel,
        mesh=mesh,
        out_type=[
            jax.ShapeDtypeStruct((BK, C), jnp.float32),
            jax.ShapeDtypeStruct((BK, K), jnp.float32),
        ],
        scratch_types=[
            pltpu.VMEM((ch,), jnp.int32),
            pltpu.VMEM((ch,), jnp.int32),
            pltpu.VMEM((ch, C), jnp.float32),
            pltpu.VMEM((ch, K), jnp.float32),
            pltpu.SemaphoreType.DMA,
            pltpu.SemaphoreType.DMA,
        ],
    )
    def run(idxn_hbm, idxa_hbm, nodes_hbm, adj_hbm, outn_hbm, outa_hbm,
            idxn_v, idxa_v, nbuf, abuf, sem_n, sem_a):
        wid = lax.axis_index("s") * nc + lax.axis_index("c")
        base = wid * rows_per_w
        for c in range(rows_per_w // ch):
            off = base + c * ch
            pltpu.sync_copy(idxn_hbm.at[pl.ds(off, ch)], idxn_v)
            pltpu.sync_copy(idxa_hbm.at[pl.ds(off, ch)], idxa_v)
            cp_n = pltpu.async_copy(nodes_hbm.at[idxn_v], nbuf, sem_n)
            cp_a = pltpu.async_copy(adj_hbm.at[idxa_v], abuf, sem_a)
            cp_n.wait()
            cp_a.wait()
            pltpu.sync_copy(nbuf, outn_hbm.at[pl.ds(off, ch)])
            pltpu.sync_copy(abuf, outa_hbm.at[pl.ds(off, ch)])

    return run(idxn, idxa, nodes_flat, adj_flat)


def kernel(nodes, adj_mat, W, b):
    B, N, C = nodes.shape
    k = N // 2
    scores = _scores_call(nodes, W)
    idxn, idxa = _topk_call(scores, scores.reshape(B, N, 1), k)
    nodes_flat = nodes.reshape(B * N, C)
    adj_flat = adj_mat.reshape(B * N * 2, k)
    nw = 32
    out_n, out_a = _gather_call(
        idxn.reshape(B * k), idxa.reshape(B * k), nodes_flat, adj_flat,
        rows_per_w=(B * k) // nw, ch=64)
    return out_n.reshape(B, k, C), out_a.reshape(B, k, k)


# speedup vs baseline: 1.0546x; 1.0546x over previous
import functools

import jax
import jax.numpy as jnp
from jax import lax
from jax.experimental import pallas as pl
from jax.experimental.pallas import tpu as pltpu
from jax.experimental.pallas import tpu_sc as plsc


def _topk_body(k, chunk, nodes_ref, w_ref, idxn_ref, idxa_ref):
    b = pl.program_id(0)
    N = nodes_ref.shape[1]
    x = nodes_ref[0]
    w = w_ref[...]
    s_row = lax.dot_general(w, x, (((0,), (1,)), ((), ())),
                            preferred_element_type=jnp.float32)
    s_col = jnp.transpose(s_row, (1, 0))
    r_iota = lax.broadcasted_iota(jnp.int32, (1, k), 1)
    j_iota = lax.broadcasted_iota(jnp.int32, (chunk, N), 1)
    idx_acc = jnp.zeros((1, k), dtype=jnp.int32)
    for ci in range(0, N, chunk):
        sc = s_col[ci:ci + chunk, :]
        i_col = ci + lax.broadcasted_iota(jnp.int32, (chunk, 1), 0)
        before = (s_row > sc) | ((s_row == sc) & (j_iota < i_col))
        rank = jnp.sum(jnp.where(before, 1.0, 0.0), axis=1,
                       keepdims=True).astype(jnp.int32)
        onehot = rank == r_iota
        contrib = jnp.where(onehot, i_col, 0)
        idx_acc = idx_acc + jnp.sum(contrib, axis=0, keepdims=True)
    idx_glob = idx_acc + b * N
    idxn_ref[0] = idx_glob
    idxa_ref[0] = idx_glob * 2


def _topk_call(nodes, W, k, chunk=256):
    B, N, C = nodes.shape
    body = functools.partial(_topk_body, k, chunk)
    return pl.pallas_call(
        body,
        grid=(B,),
        in_specs=[
            pl.BlockSpec((1, N, C), lambda b: (b, 0, 0)),
            pl.BlockSpec((C, 1), lambda b: (0, 0)),
        ],
        out_specs=[
            pl.BlockSpec((1, 1, k), lambda b: (b, 0, 0)),
            pl.BlockSpec((1, 1, k), lambda b: (b, 0, 0)),
        ],
        out_shape=[
            jax.ShapeDtypeStruct((B, 1, k), jnp.int32),
            jax.ShapeDtypeStruct((B, 1, k), jnp.int32),
        ],
    )(nodes, W)


def _gather_call(idxn, idxa, nodes_flat, adj_flat, rows_per_w, ch):
    BK = idxn.shape[0]
    C = nodes_flat.shape[1]
    K = adj_flat.shape[1]
    info = plsc.get_sparse_core_info()
    nc = info.num_cores

    mesh = plsc.VectorSubcoreMesh(core_axis_name="c", subcore_axis_name="s")

    @functools.partial(
        pl.kernel,
        mesh=mesh,
        out_type=[
            jax.ShapeDtypeStruct((BK, C), jnp.float32),
            jax.ShapeDtypeStruct((BK, K), jnp.float32),
        ],
        scratch_types=[
            pltpu.VMEM((ch,), jnp.int32),
            pltpu.VMEM((ch,), jnp.int32),
            pltpu.VMEM((ch, C), jnp.float32),
            pltpu.VMEM((ch, K), jnp.float32),
            pltpu.SemaphoreType.DMA,
            pltpu.SemaphoreType.DMA,
        ],
    )
    def run(idxn_hbm, idxa_hbm, nodes_hbm, adj_hbm, outn_hbm, outa_hbm,
            idxn_v, idxa_v, nbuf, abuf, sem_n, sem_a):
        wid = lax.axis_index("s") * nc + lax.axis_index("c")
        base = wid * rows_per_w
        for c in range(rows_per_w // ch):
            off = base + c * ch
            pltpu.sync_copy(idxn_hbm.at[pl.ds(off, ch)], idxn_v)
            pltpu.sync_copy(idxa_hbm.at[pl.ds(off, ch)], idxa_v)
            cp_n = pltpu.async_copy(nodes_hbm.at[idxn_v], nbuf, sem_n)
            cp_a = pltpu.async_copy(adj_hbm.at[idxa_v], abuf, sem_a)
            cp_n.wait()
            cp_a.wait()
            pltpu.sync_copy(nbuf, outn_hbm.at[pl.ds(off, ch)])
            pltpu.sync_copy(abuf, outa_hbm.at[pl.ds(off, ch)])

    return run(idxn, idxa, nodes_flat, adj_flat)


def kernel(nodes, adj_mat, W, b):
    B, N, C = nodes.shape
    k = N // 2
    idxn, idxa = _topk_call(nodes, W, k)
    nodes_flat = nodes.reshape(B * N, C)
    adj_flat = adj_mat.reshape(B * N * 2, k)
    nw = 32
    out_n, out_a = _gather_call(
        idxn.reshape(B * k), idxa.reshape(B * k), nodes_flat, adj_flat,
        rows_per_w=(B * k) // nw, ch=64)
    return out_n.reshape(B, k, C), out_a.reshape(B, k, k)


# speedup vs baseline: 2.3708x; 2.2480x over previous
import functools

import jax
import jax.numpy as jnp
from jax import lax
from jax.experimental import pallas as pl
from jax.experimental.pallas import tpu as pltpu
from jax.experimental.pallas import tpu_sc as plsc


def _topk_body(k, chunk, nodes_ref, w_ref, idxn_ref):
    b = pl.program_id(0)
    N = nodes_ref.shape[1]
    x = nodes_ref[0]
    w = w_ref[...]
    s_row = lax.dot_general(w, x, (((0,), (1,)), ((), ())),
                            preferred_element_type=jnp.float32)
    s_col = jnp.transpose(s_row, (1, 0))
    r_iota = lax.broadcasted_iota(jnp.int32, (1, k), 1)
    j_iota = lax.broadcasted_iota(jnp.int32, (chunk, N), 1)
    idx_acc = jnp.zeros((1, k), dtype=jnp.int32)
    for ci in range(0, N, chunk):
        sc = s_col[ci:ci + chunk, :]
        i_col = ci + lax.broadcasted_iota(jnp.int32, (chunk, 1), 0)
        before = (s_row > sc) | ((s_row == sc) & (j_iota < i_col))
        rank = jnp.sum(jnp.where(before, 1.0, 0.0), axis=1,
                       keepdims=True).astype(jnp.int32)
        onehot = rank == r_iota
        contrib = jnp.where(onehot, i_col, 0)
        idx_acc = idx_acc + jnp.sum(contrib, axis=0, keepdims=True)
    idxn_ref[0] = idx_acc + b * N


def _topk_call(nodes, W, k, chunk=256):
    B, N, C = nodes.shape
    body = functools.partial(_topk_body, k, chunk)
    return pl.pallas_call(
        body,
        grid=(B,),
        in_specs=[
            pl.BlockSpec((1, N, C), lambda b: (b, 0, 0)),
            pl.BlockSpec((C, 1), lambda b: (0, 0)),
        ],
        out_specs=pl.BlockSpec((1, 1, k), lambda b: (b, 0, 0)),
        out_shape=jax.ShapeDtypeStruct((B, 1, k), jnp.int32),
    )(nodes, W)


def _gather_call(idxn, nodes_flat, adj_flat, k, rows_per_w, ch):
    BK = idxn.shape[0]
    C = nodes_flat.shape[1]
    N = adj_flat.shape[1]
    info = plsc.get_sparse_core_info()
    nc = info.num_cores

    mesh = plsc.VectorSubcoreMesh(core_axis_name="c", subcore_axis_name="s")

    @functools.partial(
        pl.kernel,
        mesh=mesh,
        out_type=[
            jax.ShapeDtypeStruct((BK, C), jnp.float32),
            jax.ShapeDtypeStruct((BK, k), jnp.float32),
        ],
        scratch_types=[
            pltpu.VMEM((ch,), jnp.int32),
            pltpu.VMEM((ch, C), jnp.float32),
            pltpu.VMEM((ch, N), jnp.float32),
            pltpu.SemaphoreType.DMA,
            pltpu.SemaphoreType.DMA,
        ],
    )
    def run(idxn_hbm, nodes_hbm, adj_hbm, outn_hbm, outa_hbm,
            idx_v, nbuf, abuf, sem_n, sem_a):
        wid = lax.axis_index("s") * nc + lax.axis_index("c")
        base = wid * rows_per_w
        for c in range(rows_per_w // ch):
            off = base + c * ch
            pltpu.sync_copy(idxn_hbm.at[pl.ds(off, ch)], idx_v)
            cp_n = pltpu.async_copy(nodes_hbm.at[idx_v], nbuf, sem_n)
            cp_a = pltpu.async_copy(adj_hbm.at[idx_v], abuf, sem_a)
            cp_n.wait()
            cp_a.wait()
            pltpu.sync_copy(nbuf, outn_hbm.at[pl.ds(off, ch)])
            pltpu.sync_copy(abuf.at[:, pl.ds(0, k)], outa_hbm.at[pl.ds(off, ch)])

    return run(idxn, nodes_flat, adj_flat)


def kernel(nodes, adj_mat, W, b):
    B, N, C = nodes.shape
    k = N // 2
    idxn = _topk_call(nodes, W, k)
    nodes_flat = nodes.reshape(B * N, C)
    adj_flat = adj_mat.reshape(B * N, N)
    nw = 32
    out_n, out_a = _gather_call(
        idxn.reshape(B * k), nodes_flat, adj_flat, k,
        rows_per_w=(B * k) // nw, ch=32)
    return out_n.reshape(B, k, C), out_a.reshape(B, k, k)
